# baseline (speedup 1.0000x reference)
import functools

import jax
import jax.numpy as jnp
from jax import lax
from jax.experimental import pallas as pl
from jax.experimental.pallas import tpu as pltpu
from jax.experimental.pallas import tpu_sc as plsc

D_MODEL = 4096
E = 64
K = 8
BLK = 512
NC, NS, L = 2, 16, 16
NW = NC * NS


def _gate_body(t_ref, x_ref, wt_ref, b_ref, logits_ref):
    logits = jnp.dot(x_ref[...], wt_ref[...], preferred_element_type=jnp.float32)
    logits_ref[...] = (logits + b_ref[...]) / t_ref[0]


def _gate(xf, wt, b2, t, N):
    return pl.pallas_call(
        _gate_body,
        grid=(N // BLK,),
        in_specs=[
            pl.BlockSpec(memory_space=pltpu.SMEM),
            pl.BlockSpec((BLK, D_MODEL), lambda i: (i, 0)),
            pl.BlockSpec((D_MODEL, E), lambda i: (0, 0)),
            pl.BlockSpec((1, E), lambda i: (0, 0)),
        ],
        out_specs=pl.BlockSpec((BLK, E), lambda i: (i, 0)),
        out_shape=jax.ShapeDtypeStruct((N, E), jnp.float32),
    )(t, xf, wt, b2)


def _route_sc(logits_flat, N):
    rows_w = N // NW
    ngroups = rows_w // L
    mesh = plsc.VectorSubcoreMesh(core_axis_name="c", subcore_axis_name="s")

    @functools.partial(
        pl.kernel,
        mesh=mesh,
        compiler_params=pltpu.CompilerParams(needs_layout_passes=False),
        out_type=(
            jax.ShapeDtypeStruct((N * K,), jnp.int32),
            jax.ShapeDtypeStruct((NW, E * L), jnp.float32),
        ),
        scratch_types=[
            pltpu.VMEM((rows_w * E,), jnp.float32),
            pltpu.VMEM((rows_w * K,), jnp.int32),
            pltpu.VMEM((E * L,), jnp.float32),
            pltpu.SemaphoreType.DMA,
        ],
    )
    def route(logits_hbm, idx_hbm, hist_hbm, slab, idxs, hist, sem):
        wid = lax.axis_index("s") * NC + lax.axis_index("c")
        base = wid * rows_w
        cp = pltpu.async_copy(
            logits_hbm.at[pl.ds(base * E, rows_w * E)], slab, sem
        )
        zeros16 = jnp.zeros((L,), jnp.float32)
        for e in range(E):
            hist[pl.ds(e * L, L)] = zeros16
        cp.wait()
        lanes = lax.iota(jnp.int32, L)
        ones16 = jnp.ones((L,), jnp.float32)
        neginf = jnp.full((L,), -jnp.inf, jnp.float32)

        def group(g, carry):
            ridx = g * L + lanes
            ridx_e = ridx * E
            ridx_k = ridx * K
            for s in range(K):
                best_v = neginf
                best_i = jnp.zeros((L,), jnp.int32)
                for e in range(E):
                    v = plsc.load_gather(slab, [ridx_e + e])
                    gt = v > best_v
                    best_v = jnp.where(gt, v, best_v)
                    best_i = jnp.where(gt, e, best_i)
                plsc.store_scatter(idxs, [ridx_k + s], best_i)
                if s == 0:
                    plsc.addupdate_scatter(hist, [lanes * E + best_i], ones16)
                if s < K - 1:
                    plsc.store_scatter(slab, [ridx_e + best_i], neginf)
            return carry

        lax.fori_loop(0, ngroups, group, 0)
        pltpu.sync_copy(idxs, idx_hbm.at[pl.ds(base * K, rows_w * K)])
        pltpu.sync_copy(hist, hist_hbm.at[wid])

    return route(logits_flat)


def _loss_body(n_tokens, hist_ref, loss_ref):
    counts = jnp.sum(hist_ref[...], axis=0)
    actual = counts / n_tokens + 1e-8
    actual = actual / jnp.sum(actual)
    u = 1.0 / E
    kl = jnp.sum(u * (jnp.log(u) - jnp.log(actual)))
    loss_ref[...] = jnp.full((1, 1), 0.1 * kl, dtype=jnp.float32)


def _loss(hist, N):
    return pl.pallas_call(
        functools.partial(_loss_body, N),
        out_shape=jax.ShapeDtypeStruct((1, 1), jnp.float32),
    )(hist)


def kernel(x, W, b, gate_temperature):
    B, S, D = x.shape
    N = B * S
    xf = x.reshape(N, D)
    logits = _gate(xf, W.T, b.reshape(1, E), gate_temperature, N)
    idxs_flat, hist = _route_sc(logits.reshape(N * E), N)
    loss = _loss(hist.reshape(NW * L, E), N)
    return logits, idxs_flat.reshape(N, K), loss.reshape(())


# speedup vs baseline: 1.7195x; 1.7195x over previous
import functools

import jax
import jax.numpy as jnp
from jax import lax
from jax.experimental import pallas as pl
from jax.experimental.pallas import tpu as pltpu
from jax.experimental.pallas import tpu_sc as plsc

D_MODEL = 4096
E = 64
K = 8
BLK = 512
NC, NS, L = 2, 16, 16
NW = NC * NS


def _gate_body(t_ref, x_ref, wt_ref, b_ref, logits_ref):
    logits = jnp.dot(x_ref[...], wt_ref[...], preferred_element_type=jnp.float32)
    logits_ref[...] = (logits + b_ref[...]) / t_ref[0]


def _gate(xf, wt, b2, t, N):
    return pl.pallas_call(
        _gate_body,
        grid=(N // BLK,),
        in_specs=[
            pl.BlockSpec(memory_space=pltpu.SMEM),
            pl.BlockSpec((BLK, D_MODEL), lambda i: (i, 0)),
            pl.BlockSpec((D_MODEL, E), lambda i: (0, 0)),
            pl.BlockSpec((1, E), lambda i: (0, 0)),
        ],
        out_specs=pl.BlockSpec((BLK, E), lambda i: (i, 0)),
        out_shape=jax.ShapeDtypeStruct((N, E), jnp.float32),
    )(t, xf, wt, b2)


def _route_sc(logits_flat, N):
    rows_w = N // NW
    ngroups = rows_w // L
    mesh = plsc.VectorSubcoreMesh(core_axis_name="c", subcore_axis_name="s")

    @functools.partial(
        pl.kernel,
        mesh=mesh,
        compiler_params=pltpu.CompilerParams(needs_layout_passes=False),
        out_type=(
            jax.ShapeDtypeStruct((N * K,), jnp.int32),
            jax.ShapeDtypeStruct((NW, E), jnp.float32),
        ),
        scratch_types=[
            pltpu.VMEM((rows_w * E,), jnp.float32),
            pltpu.VMEM((rows_w * K + L,), jnp.int32),
            pltpu.VMEM((E,), jnp.float32),
            pltpu.SemaphoreType.DMA,
        ],
    )
    def route(logits_hbm, idx_hbm, hist_hbm, slab, idxs, hist, sem):
        wid = lax.axis_index("s") * NC + lax.axis_index("c")
        base = wid * rows_w
        cp = pltpu.async_copy(
            logits_hbm.at[pl.ds(base * E, rows_w * E)], slab, sem
        )
        zeros16 = jnp.zeros((L,), jnp.float32)
        for i in range(E // L):
            hist[pl.ds(i * L, L)] = zeros16
        cp.wait()
        lanes = lax.iota(jnp.int32, L)
        ones16 = jnp.ones((L,), jnp.float32)
        lane0 = lanes == 0
        first8 = lanes < K

        def merge(ak, ai, bk, bi):
            rbk = lax.rev(bk, (0,))
            rbi = lax.rev(bi, (0,))
            ge = ak >= rbk
            return jnp.where(ge, ak, rbk), jnp.where(ge, ai, rbi)

        def row(r, carry):
            off = r * E
            sk, si = [], []
            for q in range(4):
                kq = slab[pl.ds(off + q * L, L)]
                a, b = plsc.sort_key_val(kq, lanes + q * L, descending=True)
                sk.append(a)
                si.append(b)
            m01k, m01i = merge(sk[0], si[0], sk[1], si[1])
            m23k, m23i = merge(sk[2], si[2], sk[3], si[3])
            t01k, t01i = plsc.sort_key_val(m01k, m01i, descending=True)
            t23k, t23i = plsc.sort_key_val(m23k, m23i, descending=True)
            fk, fi = merge(t01k, t01i, t23k, t23i)
            _, top_i = plsc.sort_key_val(fk, fi, descending=True)
            plsc.store_compressed(idxs.at[pl.ds(r * K, L)], top_i, mask=first8)
            plsc.addupdate_scatter(hist, [top_i], ones16, mask=lane0)
            return carry

        lax.fori_loop(0, rows_w, row, 0)
        pltpu.sync_copy(
            idxs.at[pl.ds(0, rows_w * K)], idx_hbm.at[pl.ds(base * K, rows_w * K)]
        )
        pltpu.sync_copy(hist, hist_hbm.at[wid])

    return route(logits_flat)


def _loss_body(n_tokens, hist_ref, loss_ref):
    counts = jnp.sum(hist_ref[...], axis=0)
    actual = counts / n_tokens + 1e-8
    actual = actual / jnp.sum(actual)
    u = 1.0 / E
    kl = jnp.sum(u * (jnp.log(u) - jnp.log(actual)))
    loss_ref[...] = jnp.full((1, 1), 0.1 * kl, dtype=jnp.float32)


def _loss(hist, N):
    return pl.pallas_call(
        functools.partial(_loss_body, N),
        out_shape=jax.ShapeDtypeStruct((1, 1), jnp.float32),
    )(hist)


def kernel(x, W, b, gate_temperature):
    B, S, D = x.shape
    N = B * S
    xf = x.reshape(N, D)
    logits = _gate(xf, W.T, b.reshape(1, E), gate_temperature, N)
    idxs_flat, hist = _route_sc(logits.reshape(N * E), N)
    loss = _loss(hist, N)
    return logits, idxs_flat.reshape(N, K), loss.reshape(())


# speedup vs baseline: 1.8743x; 1.0900x over previous
import functools

import jax
import jax.numpy as jnp
from jax import lax
from jax.experimental import pallas as pl
from jax.experimental.pallas import tpu as pltpu
from jax.experimental.pallas import tpu_sc as plsc

D_MODEL = 4096
E = 64
K = 8
BLK = 512
NC, NS, L = 2, 16, 16
NW = NC * NS


def _gate_body(t_ref, x_ref, wt_ref, b_ref, logits_ref):
    logits = jnp.dot(x_ref[...], wt_ref[...], preferred_element_type=jnp.float32)
    logits_ref[...] = (logits + b_ref[...]) / t_ref[0]


def _gate(xf, wt, b2, t, N):
    return pl.pallas_call(
        _gate_body,
        grid=(N // BLK,),
        in_specs=[
            pl.BlockSpec(memory_space=pltpu.SMEM),
            pl.BlockSpec((BLK, D_MODEL), lambda i: (i, 0)),
            pl.BlockSpec((D_MODEL, E), lambda i: (0, 0)),
            pl.BlockSpec((1, E), lambda i: (0, 0)),
        ],
        out_specs=pl.BlockSpec((BLK, E), lambda i: (i, 0)),
        out_shape=jax.ShapeDtypeStruct((N, E), jnp.float32),
    )(t, xf, wt, b2)


def _route_sc(logits_flat, N):
    rows_w = N // NW
    ngroups = rows_w // L
    mesh = plsc.VectorSubcoreMesh(core_axis_name="c", subcore_axis_name="s")

    @functools.partial(
        pl.kernel,
        mesh=mesh,
        compiler_params=pltpu.CompilerParams(needs_layout_passes=False),
        out_type=(
            jax.ShapeDtypeStruct((N * K,), jnp.int32),
            jax.ShapeDtypeStruct((NW, E), jnp.float32),
        ),
        scratch_types=[
            pltpu.VMEM((rows_w * E,), jnp.float32),
            pltpu.VMEM((rows_w * K + L,), jnp.int32),
            pltpu.VMEM((E,), jnp.float32),
            pltpu.SemaphoreType.DMA,
        ],
    )
    def route(logits_hbm, idx_hbm, hist_hbm, slab, idxs, hist, sem):
        wid = lax.axis_index("s") * NC + lax.axis_index("c")
        base = wid * rows_w
        cp = pltpu.async_copy(
            logits_hbm.at[pl.ds(base * E, rows_w * E)], slab, sem
        )
        zeros16 = jnp.zeros((L,), jnp.float32)
        for i in range(E // L):
            hist[pl.ds(i * L, L)] = zeros16
        cp.wait()
        lanes = lax.iota(jnp.int32, L)
        ones16 = jnp.ones((L,), jnp.float32)
        lane0 = lanes == 0
        first8 = lanes < K

        def merge(ak, ai, bk, bi):
            rbk = lax.rev(bk, (0,))
            rbi = lax.rev(bi, (0,))
            ge = ak >= rbk
            return jnp.where(ge, ak, rbk), jnp.where(ge, ai, rbi)

        def top8(off):
            sk, si = [], []
            for q in range(4):
                kq = slab[pl.ds(off + q * L, L)]
                a, b = plsc.sort_key_val(kq, lanes + q * L, descending=True)
                sk.append(a)
                si.append(b)
            m01k, m01i = merge(sk[0], si[0], sk[1], si[1])
            m23k, m23i = merge(sk[2], si[2], sk[3], si[3])
            t01k, t01i = plsc.sort_key_val(m01k, m01i, descending=True)
            t23k, t23i = plsc.sort_key_val(m23k, m23i, descending=True)
            fk, fi = merge(t01k, t01i, t23k, t23i)
            _, top_i = plsc.sort_key_val(fk, fi, descending=True)
            return top_i

        UNROLL = 4

        def rows(g, carry):
            r = g * UNROLL
            tops = [top8((r + u) * E) for u in range(UNROLL)]
            for u in range(UNROLL):
                plsc.store_compressed(
                    idxs.at[pl.ds((r + u) * K, L)], tops[u], mask=first8
                )
                plsc.addupdate_scatter(hist, [tops[u]], ones16, mask=lane0)
            return carry

        lax.fori_loop(0, rows_w // UNROLL, rows, 0)
        pltpu.sync_copy(
            idxs.at[pl.ds(0, rows_w * K)], idx_hbm.at[pl.ds(base * K, rows_w * K)]
        )
        pltpu.sync_copy(hist, hist_hbm.at[wid])

    return route(logits_flat)


def _loss_body(n_tokens, hist_ref, loss_ref):
    counts = jnp.sum(hist_ref[...], axis=0)
    actual = counts / n_tokens + 1e-8
    actual = actual / jnp.sum(actual)
    u = 1.0 / E
    kl = jnp.sum(u * (jnp.log(u) - jnp.log(actual)))
    loss_ref[...] = jnp.full((1, 1), 0.1 * kl, dtype=jnp.float32)


def _loss(hist, N):
    return pl.pallas_call(
        functools.partial(_loss_body, N),
        out_shape=jax.ShapeDtypeStruct((1, 1), jnp.float32),
    )(hist)


def kernel(x, W, b, gate_temperature):
    B, S, D = x.shape
    N = B * S
    xf = x.reshape(N, D)
    logits = _gate(xf, W.T, b.reshape(1, E), gate_temperature, N)
    idxs_flat, hist = _route_sc(logits.reshape(N * E), N)
    loss = _loss(hist, N)
    return logits, idxs_flat.reshape(N, K), loss.reshape(())


# speedup vs baseline: 1.8983x; 1.0128x over previous
import functools

import jax
import jax.numpy as jnp
from jax import lax
from jax.experimental import pallas as pl
from jax.experimental.pallas import tpu as pltpu
from jax.experimental.pallas import tpu_sc as plsc

D_MODEL = 4096
E = 64
K = 8
BLK = 512
NC, NS, L = 2, 16, 16
NW = NC * NS


def _gate_body(t_ref, x_ref, wt_ref, b_ref, logits_ref):
    logits = jnp.dot(x_ref[...], wt_ref[...], preferred_element_type=jnp.float32)
    logits_ref[...] = (logits + b_ref[...]) / t_ref[0]


def _gate(xf, wt, b2, t, N):
    return pl.pallas_call(
        _gate_body,
        grid=(N // BLK,),
        in_specs=[
            pl.BlockSpec(memory_space=pltpu.SMEM),
            pl.BlockSpec((BLK, D_MODEL), lambda i: (i, 0)),
            pl.BlockSpec((D_MODEL, E), lambda i: (0, 0)),
            pl.BlockSpec((1, E), lambda i: (0, 0)),
        ],
        out_specs=pl.BlockSpec((BLK, E), lambda i: (i, 0)),
        out_shape=jax.ShapeDtypeStruct((N, E), jnp.float32),
    )(t, xf, wt, b2)


def _route_sc(logits_flat, N):
    rows_w = N // NW
    ngroups = rows_w // L
    mesh = plsc.VectorSubcoreMesh(core_axis_name="c", subcore_axis_name="s")

    @functools.partial(
        pl.kernel,
        mesh=mesh,
        compiler_params=pltpu.CompilerParams(needs_layout_passes=False),
        out_type=(
            jax.ShapeDtypeStruct((N * K,), jnp.int32),
            jax.ShapeDtypeStruct((NW, E), jnp.float32),
        ),
        scratch_types=[
            pltpu.VMEM((rows_w * E,), jnp.float32),
            pltpu.VMEM((rows_w * K + L,), jnp.int32),
            pltpu.VMEM((E,), jnp.float32),
            pltpu.SemaphoreType.DMA,
        ],
    )
    def route(logits_hbm, idx_hbm, hist_hbm, slab, idxs, hist, sem):
        wid = lax.axis_index("s") * NC + lax.axis_index("c")
        base = wid * rows_w
        cp = pltpu.async_copy(
            logits_hbm.at[pl.ds(base * E, rows_w * E)], slab, sem
        )
        zeros16 = jnp.zeros((L,), jnp.float32)
        for i in range(E // L):
            hist[pl.ds(i * L, L)] = zeros16
        cp.wait()
        lanes = lax.iota(jnp.int32, L)
        ones16 = jnp.ones((L,), jnp.float32)
        lane0 = lanes == 0
        first8 = lanes < K

        def merge(ak, ai, bk, bi):
            rbk = lax.rev(bk, (0,))
            rbi = lax.rev(bi, (0,))
            ge = ak >= rbk
            return jnp.where(ge, ak, rbk), jnp.where(ge, ai, rbi)

        def top8(off):
            sk, si = [], []
            for q in range(4):
                kq = slab[pl.ds(off + q * L, L)]
                a, b = plsc.sort_key_val(kq, lanes + q * L, descending=True)
                sk.append(a)
                si.append(b)
            m01k, m01i = merge(sk[0], si[0], sk[1], si[1])
            m23k, m23i = merge(sk[2], si[2], sk[3], si[3])
            t01k, t01i = plsc.sort_key_val(m01k, m01i, descending=True)
            t23k, t23i = plsc.sort_key_val(m23k, m23i, descending=True)
            fk, fi = merge(t01k, t01i, t23k, t23i)
            _, top_i = plsc.sort_key_val(fk, fi, descending=True)
            return top_i

        UNROLL = 8

        def rows(g, carry):
            r = g * UNROLL
            tops = [top8((r + u) * E) for u in range(UNROLL)]
            for u in range(UNROLL):
                plsc.store_compressed(
                    idxs.at[pl.ds((r + u) * K, L)], tops[u], mask=first8
                )
                plsc.addupdate_scatter(hist, [tops[u]], ones16, mask=lane0)
            return carry

        lax.fori_loop(0, rows_w // UNROLL, rows, 0)
        pltpu.sync_copy(
            idxs.at[pl.ds(0, rows_w * K)], idx_hbm.at[pl.ds(base * K, rows_w * K)]
        )
        pltpu.sync_copy(hist, hist_hbm.at[wid])

    return route(logits_flat)


def _loss_body(n_tokens, hist_ref, loss_ref):
    counts = jnp.sum(hist_ref[...], axis=0)
    actual = counts / n_tokens + 1e-8
    actual = actual / jnp.sum(actual)
    u = 1.0 / E
    kl = jnp.sum(u * (jnp.log(u) - jnp.log(actual)))
    loss_ref[...] = jnp.full((1, 1), 0.1 * kl, dtype=jnp.float32)


def _loss(hist, N):
    return pl.pallas_call(
        functools.partial(_loss_body, N),
        out_shape=jax.ShapeDtypeStruct((1, 1), jnp.float32),
    )(hist)


def kernel(x, W, b, gate_temperature):
    B, S, D = x.shape
    N = B * S
    xf = x.reshape(N, D)
    logits = _gate(xf, W.T, b.reshape(1, E), gate_temperature, N)
    idxs_flat, hist = _route_sc(logits.reshape(N * E), N)
    loss = _loss(hist, N)
    return logits, idxs_flat.reshape(N, K), loss.reshape(())
